# Initial kernel scaffold; baseline (speedup 1.0000x reference)
import jax
import jax.numpy as jnp
from jax.experimental import pallas as pl


def kernel(embedding_0, batch, lin_w, lin_b, nl_w1, nl_b1, nl_w2, nl_b2, mlp_w1, mlp_b1, mlp_w2, mlp_b2):
    raise NotImplementedError("write your pallas kernel here")


# speedup vs baseline: 7.5554x; 7.5554x over previous
import jax
import jax.numpy as jnp
from jax.experimental import pallas as pl

_QS = [0.1, 0.2, 0.3, 0.4, 0.5, 0.6, 0.7, 0.8, 0.9]
_ROWS = 2048
_NSEG = 64


def _node_kernel(x_ref, wcf_ref, w1e_ref, b1_ref, w2_ref, c0_ref, out_ref):
    x = x_ref[...]
    lin = jnp.sum(x * wcf_ref[...], axis=1)
    h = jnp.dot(x, w1e_ref[...], preferred_element_type=jnp.float32)
    h = h + b1_ref[...]
    h = h * jax.nn.sigmoid(h)
    nlv = jnp.sum(h * w2_ref[...], axis=1)
    out_ref[0, 0, :] = lin + nlv + c0_ref[0, 0]


def _gather_rows(s, gi, iot_r, iot_l):
    r = gi // 128
    l = gi - r * 128
    p = jnp.where(iot_r == r[:, None], 1.0, 0.0)
    g = jnp.dot(p, s, preferred_element_type=jnp.float32)
    return jnp.sum(jnp.where(iot_l == l[:, None], g, 0.0), axis=1)


def _readout_kernel(v_ref, b_ref, s_ref, qs_ref, w1_ref, b1_ref, w2_ref,
                    b2_ref, o_ref):
    rows = v_ref.shape[0]
    ch = 8
    segs3 = jax.lax.broadcasted_iota(jnp.int32, (ch, 128, _NSEG), 2)

    def body(k, carry):
        cnt, sm = carry
        bc = b_ref[pl.ds(k * ch, ch), :]
        vc = v_ref[pl.ds(k * ch, ch), :]
        oh = bc[:, :, None] == segs3
        cnt = cnt + jnp.sum(jnp.where(oh, 1.0, 0.0), axis=(0, 1))
        sm = sm + jnp.sum(jnp.where(oh, vc[:, :, None], 0.0), axis=(0, 1))
        return cnt, sm

    cnt, sm = jax.lax.fori_loop(
        0, rows // ch,
        body,
        (jnp.zeros((_NSEG,), jnp.float32), jnp.zeros((_NSEG,), jnp.float32)))
    mean = sm / jnp.maximum(cnt, 1.0)

    ii = jax.lax.broadcasted_iota(jnp.int32, (_NSEG, _NSEG), 0)
    jj = jax.lax.broadcasted_iota(jnp.int32, (_NSEG, _NSEG), 1)
    off = jnp.sum(jnp.where(ii < jj, cnt[:, None], 0.0), axis=0)

    s = s_ref[...]
    iot_r = jax.lax.broadcasted_iota(jnp.int32, (_NSEG, rows), 1)
    iot_l = jax.lax.broadcasted_iota(jnp.int32, (_NSEG, 128), 1)
    offi = off.astype(jnp.int32)
    cnti = cnt.astype(jnp.int32)
    mx = _gather_rows(s, offi + jnp.maximum(cnti - 1, 0), iot_r, iot_l)
    mn = _gather_rows(s, offi, iot_r, iot_l)

    qs = qs_ref[0, :]
    pos = qs[None, :] * (cnt[:, None] - 1.0)
    lo = jnp.floor(pos)
    hi = jnp.ceil(pos)
    frac = pos - lo
    cols = [mean, mx, mn]
    for q in range(9):
        gl = (off + lo[:, q]).astype(jnp.int32)
        gh = (off + hi[:, q]).astype(jnp.int32)
        vlo = _gather_rows(s, gl, iot_r, iot_l)
        vhi = _gather_rows(s, gh, iot_r, iot_l)
        f = frac[:, q]
        cols.append(vlo * (1.0 - f) + vhi * f)
    cols += [jnp.zeros((_NSEG,), jnp.float32)] * 4
    agg = jnp.stack(cols, axis=1)
    h2 = jnp.dot(agg, w1_ref[...], preferred_element_type=jnp.float32)
    h2 = h2 + b1_ref[...]
    h2 = h2 * jax.nn.sigmoid(h2)
    o_ref[0, :] = jnp.sum(h2 * w2_ref[...], axis=1) + b2_ref[0, 0]


def kernel(embedding_0, batch, lin_w, lin_b, nl_w1, nl_b1, nl_w2, nl_b2,
           mlp_w1, mlp_b1, mlp_w2, mlp_b2):
    n, ic, _, nr = embedding_0.shape
    hc = nl_w1.shape[1]
    r = _ROWS
    nblk = -(-n // r)
    np_ = nblk * r
    e2 = embedding_0.reshape(n, ic * nr)
    wp = jnp.concatenate([lin_w, jnp.zeros((1, ic), lin_w.dtype)], axis=0)
    wcf = wp.T.reshape(1, ic * nr)
    w1e = jnp.zeros((ic * nr, hc), nl_w1.dtype).at[nr - 1::nr, :].set(nl_w1)
    c0 = (jnp.sum(lin_b) + nl_b2).reshape(1, 1)

    node = pl.pallas_call(
        _node_kernel,
        grid=(nblk,),
        in_specs=[
            pl.BlockSpec((r, ic * nr), lambda i: (i, 0)),
            pl.BlockSpec((1, ic * nr), lambda i: (0, 0)),
            pl.BlockSpec((ic * nr, hc), lambda i: (0, 0)),
            pl.BlockSpec((1, hc), lambda i: (0, 0)),
            pl.BlockSpec((1, hc), lambda i: (0, 0)),
            pl.BlockSpec((1, 1), lambda i: (0, 0)),
        ],
        out_specs=pl.BlockSpec((1, 1, r), lambda i: (i, 0, 0)),
        out_shape=jax.ShapeDtypeStruct((nblk, 1, r), jnp.float32),
    )(e2, wcf, w1e, nl_b1.reshape(1, hc), nl_w2.reshape(1, hc), c0)

    node = node.reshape(np_)
    idx = jax.lax.iota(jnp.int32, np_)
    node = jnp.where(idx < n, node, 0.0)
    batch_p = jnp.concatenate(
        [batch, jnp.full((np_ - n,), _NSEG, jnp.int32)])
    _, sv = jax.lax.sort((batch_p, node), num_keys=2)

    rows = np_ // 128
    qsrow = jnp.zeros((1, 128), jnp.float32).at[0, :9].set(
        jnp.array(_QS, jnp.float32))
    w1p = jnp.zeros((16, hc), mlp_w1.dtype).at[:12, :].set(mlp_w1)
    out = pl.pallas_call(
        _readout_kernel,
        grid=(1,),
        in_specs=[
            pl.BlockSpec((rows, 128), lambda i: (0, 0)),
            pl.BlockSpec((rows, 128), lambda i: (0, 0)),
            pl.BlockSpec((rows, 128), lambda i: (0, 0)),
            pl.BlockSpec((1, 128), lambda i: (0, 0)),
            pl.BlockSpec((16, hc), lambda i: (0, 0)),
            pl.BlockSpec((1, hc), lambda i: (0, 0)),
            pl.BlockSpec((1, hc), lambda i: (0, 0)),
            pl.BlockSpec((1, 1), lambda i: (0, 0)),
        ],
        out_specs=pl.BlockSpec((1, _NSEG), lambda i: (0, 0)),
        out_shape=jax.ShapeDtypeStruct((1, _NSEG), jnp.float32),
    )(node.reshape(rows, 128), batch_p.reshape(rows, 128),
      sv.reshape(rows, 128), qsrow, w1p, mlp_b1.reshape(1, hc),
      mlp_w2.reshape(1, hc), mlp_b2.reshape(1, 1))
    return out[0]


# speedup vs baseline: 15.8616x; 2.0994x over previous
import jax
import jax.numpy as jnp
from jax.experimental import pallas as pl

_QS = [0.1, 0.2, 0.3, 0.4, 0.5, 0.6, 0.7, 0.8, 0.9]
_ROWS = 2048
_NSEG = 64


def _node_kernel(x_ref, wcf_ref, w1e_ref, b1_ref, w2_ref, c0_ref, out_ref):
    x = x_ref[...]
    ic = w1e_ref.shape[0]
    lin = jnp.sum(x * wcf_ref[...], axis=1)
    h = jnp.dot(x[:, x.shape[1] - ic:], w1e_ref[...],
                preferred_element_type=jnp.float32)
    h = h + b1_ref[...]
    h = h * jax.nn.sigmoid(h)
    nlv = jnp.sum(h * w2_ref[...], axis=1)
    out_ref[0, 0, :] = lin + nlv + c0_ref[0, 0]


def _gather_rows(s, gi, iot_r, iot_l):
    r = gi // 128
    l = gi - r * 128
    p = jnp.where(iot_r == r[:, None], 1.0, 0.0)
    g = jnp.dot(p, s, preferred_element_type=jnp.float32)
    return jnp.sum(jnp.where(iot_l == l[:, None], g, 0.0), axis=1)


def _readout_kernel(v_ref, b_ref, s_ref, qs_ref, w1_ref, b1_ref, w2_ref,
                    b2_ref, o_ref):
    rows = v_ref.shape[0]
    ch = 8
    segs3 = jax.lax.broadcasted_iota(jnp.int32, (ch, 128, _NSEG), 2)

    def body(k, carry):
        cnt, sm = carry
        bc = b_ref[pl.ds(k * ch, ch), :]
        vc = v_ref[pl.ds(k * ch, ch), :]
        oh = bc[:, :, None] == segs3
        cnt = cnt + jnp.sum(jnp.where(oh, 1.0, 0.0), axis=(0, 1))
        sm = sm + jnp.sum(jnp.where(oh, vc[:, :, None], 0.0), axis=(0, 1))
        return cnt, sm

    cnt, sm = jax.lax.fori_loop(
        0, rows // ch,
        body,
        (jnp.zeros((_NSEG,), jnp.float32), jnp.zeros((_NSEG,), jnp.float32)))
    mean = sm / jnp.maximum(cnt, 1.0)

    ii = jax.lax.broadcasted_iota(jnp.int32, (_NSEG, _NSEG), 0)
    jj = jax.lax.broadcasted_iota(jnp.int32, (_NSEG, _NSEG), 1)
    off = jnp.sum(jnp.where(ii < jj, cnt[:, None], 0.0), axis=0)

    s = s_ref[...]
    iot_r = jax.lax.broadcasted_iota(jnp.int32, (_NSEG, rows), 1)
    iot_l = jax.lax.broadcasted_iota(jnp.int32, (_NSEG, 128), 1)
    offi = off.astype(jnp.int32)
    cnti = cnt.astype(jnp.int32)
    mx = _gather_rows(s, offi + jnp.maximum(cnti - 1, 0), iot_r, iot_l)
    mn = _gather_rows(s, offi, iot_r, iot_l)

    qs = qs_ref[0, :]
    pos = qs[None, :] * (cnt[:, None] - 1.0)
    lo = jnp.floor(pos)
    hi = jnp.ceil(pos)
    frac = pos - lo
    cols = [mean, mx, mn]
    for q in range(9):
        gl = (off + lo[:, q]).astype(jnp.int32)
        gh = (off + hi[:, q]).astype(jnp.int32)
        vlo = _gather_rows(s, gl, iot_r, iot_l)
        vhi = _gather_rows(s, gh, iot_r, iot_l)
        f = frac[:, q]
        cols.append(vlo * (1.0 - f) + vhi * f)
    cols += [jnp.zeros((_NSEG,), jnp.float32)] * 4
    agg = jnp.stack(cols, axis=1)
    h2 = jnp.dot(agg, w1_ref[...], preferred_element_type=jnp.float32)
    h2 = h2 + b1_ref[...]
    h2 = h2 * jax.nn.sigmoid(h2)
    o_ref[0, :] = jnp.sum(h2 * w2_ref[...], axis=1) + b2_ref[0, 0]


def kernel(embedding_0, batch, lin_w, lin_b, nl_w1, nl_b1, nl_w2, nl_b2,
           mlp_w1, mlp_b1, mlp_w2, mlp_b2):
    n, ic, _, nr = embedding_0.shape
    hc = nl_w1.shape[1]
    r = _ROWS
    nblk = -(-n // r)
    np_ = nblk * r
    e2 = embedding_0.transpose(0, 3, 1, 2).reshape(n, nr * ic)
    wcf = jnp.concatenate(
        [lin_w.reshape(1, (nr - 1) * ic), jnp.zeros((1, ic), lin_w.dtype)],
        axis=1)
    c0 = (jnp.sum(lin_b) + nl_b2).reshape(1, 1)

    node = pl.pallas_call(
        _node_kernel,
        grid=(nblk,),
        in_specs=[
            pl.BlockSpec((r, ic * nr), lambda i: (i, 0)),
            pl.BlockSpec((1, ic * nr), lambda i: (0, 0)),
            pl.BlockSpec((ic, hc), lambda i: (0, 0)),
            pl.BlockSpec((1, hc), lambda i: (0, 0)),
            pl.BlockSpec((1, hc), lambda i: (0, 0)),
            pl.BlockSpec((1, 1), lambda i: (0, 0)),
        ],
        out_specs=pl.BlockSpec((1, 1, r), lambda i: (i, 0, 0)),
        out_shape=jax.ShapeDtypeStruct((nblk, 1, r), jnp.float32),
    )(e2, wcf, nl_w1, nl_b1.reshape(1, hc), nl_w2.reshape(1, hc), c0)

    node = node.reshape(np_)
    idx = jax.lax.iota(jnp.int32, np_)
    node = jnp.where(idx < n, node, 0.0)
    batch_p = jnp.concatenate(
        [batch, jnp.full((np_ - n,), _NSEG, jnp.int32)])
    _, sv = jax.lax.sort((batch_p, node), num_keys=2)

    rows = np_ // 128
    qsrow = jnp.zeros((1, 128), jnp.float32).at[0, :9].set(
        jnp.array(_QS, jnp.float32))
    w1p = jnp.zeros((16, hc), mlp_w1.dtype).at[:12, :].set(mlp_w1)
    out = pl.pallas_call(
        _readout_kernel,
        grid=(1,),
        in_specs=[
            pl.BlockSpec((rows, 128), lambda i: (0, 0)),
            pl.BlockSpec((rows, 128), lambda i: (0, 0)),
            pl.BlockSpec((rows, 128), lambda i: (0, 0)),
            pl.BlockSpec((1, 128), lambda i: (0, 0)),
            pl.BlockSpec((16, hc), lambda i: (0, 0)),
            pl.BlockSpec((1, hc), lambda i: (0, 0)),
            pl.BlockSpec((1, hc), lambda i: (0, 0)),
            pl.BlockSpec((1, 1), lambda i: (0, 0)),
        ],
        out_specs=pl.BlockSpec((1, _NSEG), lambda i: (0, 0)),
        out_shape=jax.ShapeDtypeStruct((1, _NSEG), jnp.float32),
    )(node.reshape(rows, 128), batch_p.reshape(rows, 128),
      sv.reshape(rows, 128), qsrow, w1p, mlp_b1.reshape(1, hc),
      mlp_w2.reshape(1, hc), mlp_b2.reshape(1, 1))
    return out[0]
